# Initial kernel scaffold; baseline (speedup 1.0000x reference)
import jax
import jax.numpy as jnp
from jax.experimental import pallas as pl


def kernel(node_embeddings, edge_index, W_q, W_k):
    raise NotImplementedError("write your pallas kernel here")


# speedup vs baseline: 1.8072x; 1.8072x over previous
import functools

import jax
import jax.numpy as jnp
from jax import lax
from jax.experimental import pallas as pl
from jax.experimental.pallas import tpu as pltpu
from jax.experimental.pallas import tpu_sc as plsc

_EMBED = 128
_ADIM = 64
_NC, _NS, _L = 2, 16, 16
_NW = _NC * _NS
_CHUNK = 80


@functools.lru_cache(maxsize=None)
def _make_project(n_nodes: int, block: int):
    def body(x_ref, wq_ref, wk_ref, q_ref, k_ref):
        x = x_ref[...]
        ssq = jnp.sum(x * x, axis=1, keepdims=True)
        inv = 1.0 / jnp.maximum(jnp.sqrt(ssq), 1e-12)
        xn = x * inv
        scale = 1.0 / (_ADIM ** 0.5)
        q_ref[...] = lax.dot(xn, wq_ref[...], preferred_element_type=jnp.float32) * scale
        k_ref[...] = lax.dot(xn, wk_ref[...], preferred_element_type=jnp.float32)

    return pl.pallas_call(
        body,
        grid=(n_nodes // block,),
        in_specs=[
            pl.BlockSpec((block, _EMBED), lambda i: (i, 0)),
            pl.BlockSpec((_EMBED, _ADIM), lambda i: (0, 0)),
            pl.BlockSpec((_EMBED, _ADIM), lambda i: (0, 0)),
        ],
        out_specs=[
            pl.BlockSpec((block, _ADIM), lambda i: (i, 0)),
            pl.BlockSpec((block, _ADIM), lambda i: (i, 0)),
        ],
        out_shape=[
            jax.ShapeDtypeStruct((n_nodes, _ADIM), jnp.float32),
            jax.ShapeDtypeStruct((n_nodes, _ADIM), jnp.float32),
        ],
    )


@functools.lru_cache(maxsize=None)
def _make_edge_scores(n_edges: int):
    per_w = n_edges // _NW
    n_chunks = per_w // _CHUNK
    assert per_w % _CHUNK == 0 and per_w % 8 == 0

    mesh = plsc.VectorSubcoreMesh(
        core_axis_name="c", subcore_axis_name="s",
        num_cores=_NC, num_subcores=_NS,
    )

    @functools.partial(
        pl.kernel,
        out_type=jax.ShapeDtypeStruct((n_edges,), jnp.float32),
        mesh=mesh,
        scratch_types=[
            pltpu.VMEM((_CHUNK,), jnp.int32),
            pltpu.VMEM((_CHUNK,), jnp.int32),
            pltpu.VMEM((_CHUNK, _ADIM), jnp.float32),
            pltpu.VMEM((_CHUNK, _ADIM), jnp.float32),
            pltpu.VMEM((_CHUNK,), jnp.float32),
            pltpu.SemaphoreType.DMA,
            pltpu.SemaphoreType.DMA,
        ],
        compiler_params=pltpu.CompilerParams(
            needs_layout_passes=False, use_tc_tiling_on_sc=False),
    )
    def edge_scores(q_hbm, k_hbm, src_hbm, tgt_hbm, out_hbm,
                    src_v, tgt_v, qr_v, kr_v, sc_v, sem_q, sem_k):
        wid = lax.axis_index("s") * _NC + lax.axis_index("c")
        w_base = wid * per_w

        def chunk_body(c, carry):
            base = w_base + c * _CHUNK
            pltpu.sync_copy(src_hbm.at[pl.ds(base, _CHUNK)], src_v)
            pltpu.sync_copy(tgt_hbm.at[pl.ds(base, _CHUNK)], tgt_v)
            cq = pltpu.async_copy(q_hbm.at[src_v], qr_v, sem_q)
            ck = pltpu.async_copy(k_hbm.at[tgt_v], kr_v, sem_k)
            cq.wait()
            ck.wait()
            for g in range(_CHUNK // _L):
                rows = lax.iota(jnp.int32, _L) + g * _L

                def dot_body(d, acc):
                    cols = jnp.full((_L,), d, jnp.int32)
                    qv = plsc.load_gather(qr_v, [rows, cols])
                    kv = plsc.load_gather(kr_v, [rows, cols])
                    return acc + qv * kv

                acc = lax.fori_loop(0, _ADIM, dot_body,
                                    jnp.zeros((_L,), jnp.float32))
                sc_v[pl.ds(g * _L, _L)] = acc
            pltpu.sync_copy(sc_v, out_hbm.at[pl.ds(base, _CHUNK)])
            return carry

        lax.fori_loop(0, n_chunks, chunk_body, 0)

    return edge_scores


def kernel(node_embeddings, edge_index, W_q, W_k):
    n_nodes = node_embeddings.shape[0]
    n_edges = edge_index.shape[1]
    project = _make_project(n_nodes, 2000)
    q_tab, k_tab = project(node_embeddings, W_q.T, W_k.T)
    edge_scores = _make_edge_scores(n_edges)
    ei = edge_index.astype(jnp.int32)
    return edge_scores(q_tab, k_tab, ei[0], ei[1])


# speedup vs baseline: 2.4044x; 1.3305x over previous
import functools

import jax
import jax.numpy as jnp
from jax import lax
from jax.experimental import pallas as pl
from jax.experimental.pallas import tpu as pltpu
from jax.experimental.pallas import tpu_sc as plsc

_EMBED = 128
_ADIM = 64
_NC, _NS, _L = 2, 16, 16
_NW = _NC * _NS
_CHUNK = 400


@functools.lru_cache(maxsize=None)
def _make_project(n_nodes: int, block: int):
    def body(x_ref, wq_ref, wk_ref, q_ref, k_ref):
        x = x_ref[...]
        ssq = jnp.sum(x * x, axis=1, keepdims=True)
        inv = 1.0 / jnp.maximum(jnp.sqrt(ssq), 1e-12)
        xn = x * inv
        scale = 1.0 / (_ADIM ** 0.5)
        q_ref[...] = lax.dot(xn, wq_ref[...], preferred_element_type=jnp.float32) * scale
        k_ref[...] = lax.dot(xn, wk_ref[...], preferred_element_type=jnp.float32)

    return pl.pallas_call(
        body,
        grid=(n_nodes // block,),
        in_specs=[
            pl.BlockSpec((block, _EMBED), lambda i: (i, 0)),
            pl.BlockSpec((_EMBED, _ADIM), lambda i: (0, 0)),
            pl.BlockSpec((_EMBED, _ADIM), lambda i: (0, 0)),
        ],
        out_specs=[
            pl.BlockSpec((block, _ADIM), lambda i: (i, 0)),
            pl.BlockSpec((block, _ADIM), lambda i: (i, 0)),
        ],
        out_shape=[
            jax.ShapeDtypeStruct((n_nodes, _ADIM), jnp.float32),
            jax.ShapeDtypeStruct((n_nodes, _ADIM), jnp.float32),
        ],
    )


@functools.lru_cache(maxsize=None)
def _make_edge_scores(n_edges: int):
    per_w = n_edges // _NW
    n_chunks = per_w // _CHUNK
    assert per_w % _CHUNK == 0 and per_w % 8 == 0

    mesh = plsc.VectorSubcoreMesh(
        core_axis_name="c", subcore_axis_name="s",
        num_cores=_NC, num_subcores=_NS,
    )

    @functools.partial(
        pl.kernel,
        out_type=jax.ShapeDtypeStruct((n_edges,), jnp.float32),
        mesh=mesh,
        scratch_types=[
            pltpu.VMEM((2, _CHUNK), jnp.int32),
            pltpu.VMEM((2, _CHUNK), jnp.int32),
            pltpu.VMEM((_CHUNK, _ADIM), jnp.float32),
            pltpu.VMEM((_CHUNK, _ADIM), jnp.float32),
            pltpu.VMEM((_CHUNK, _ADIM), jnp.float32),
            pltpu.VMEM((_CHUNK, _ADIM), jnp.float32),
            pltpu.VMEM((2, _CHUNK), jnp.float32),
            pltpu.SemaphoreType.DMA,
            pltpu.SemaphoreType.DMA,
            pltpu.SemaphoreType.DMA,
            pltpu.SemaphoreType.DMA,
            pltpu.SemaphoreType.DMA,
            pltpu.SemaphoreType.DMA,
        ],
        compiler_params=pltpu.CompilerParams(
            needs_layout_passes=False, use_tc_tiling_on_sc=False),
    )
    def edge_scores(q_hbm, k_hbm, src_hbm, tgt_hbm, out_hbm,
                    src_v, tgt_v, qr0, qr1, kr0, kr1, sc_v,
                    si0, si1, sg0, sg1, so0, so1):
        qr = [qr0, qr1]
        kr = [kr0, kr1]
        sem_i = [si0, si1]
        sem_g = [sg0, sg1]
        sem_o = [so0, so1]
        wid = lax.axis_index("s") * _NC + lax.axis_index("c")
        w_base = wid * per_w

        def idx_start(c, b):
            base = w_base + c * _CHUNK
            pltpu.async_copy(src_hbm.at[pl.ds(base, _CHUNK)], src_v.at[b], sem_i[b])
            pltpu.async_copy(tgt_hbm.at[pl.ds(base, _CHUNK)], tgt_v.at[b], sem_i[b])

        def idx_wait(b):
            pltpu.make_async_copy(src_hbm.at[pl.ds(0, _CHUNK)], src_v.at[b], sem_i[b]).wait()
            pltpu.make_async_copy(tgt_hbm.at[pl.ds(0, _CHUNK)], tgt_v.at[b], sem_i[b]).wait()

        def gather_start(b):
            pltpu.async_copy(q_hbm.at[src_v.at[b]], qr[b], sem_g[b])
            pltpu.async_copy(k_hbm.at[tgt_v.at[b]], kr[b], sem_g[b])

        def gather_wait(b):
            pltpu.make_async_copy(q_hbm.at[src_v.at[b]], qr[b], sem_g[b]).wait()
            pltpu.make_async_copy(k_hbm.at[tgt_v.at[b]], kr[b], sem_g[b]).wait()

        def out_start(c, b):
            base = w_base + c * _CHUNK
            pltpu.async_copy(sc_v.at[b], out_hbm.at[pl.ds(base, _CHUNK)], sem_o[b])

        def out_wait(b):
            pltpu.make_async_copy(sc_v.at[b], out_hbm.at[pl.ds(0, _CHUNK)], sem_o[b]).wait()

        def compute(b):
            for g in range(_CHUNK // _L):
                rows = lax.iota(jnp.int32, _L) + g * _L

                def dot_body(d, acc):
                    cols = jnp.full((_L,), d, jnp.int32)
                    qv = plsc.load_gather(qr[b], [rows, cols])
                    kv = plsc.load_gather(kr[b], [rows, cols])
                    return acc + qv * kv

                acc = lax.fori_loop(0, _ADIM, dot_body,
                                    jnp.zeros((_L,), jnp.float32), unroll=8)
                sc_v[b, pl.ds(g * _L, _L)] = acc

        idx_start(0, 0)
        idx_start(1, 1)
        idx_wait(0)
        gather_start(0)

        def pair_body(p, carry):
            for b in (0, 1):
                c = p * 2 + b

                @pl.when(c < n_chunks)
                def _():
                    @pl.when(c + 1 < n_chunks)
                    def _():
                        idx_wait(1 - b)
                        gather_start(1 - b)

                    gather_wait(b)

                    @pl.when(c + 2 < n_chunks)
                    def _():
                        idx_start(c + 2, b)

                    @pl.when(c >= 2)
                    def _():
                        out_wait(b)

                    compute(b)
                    out_start(c, b)
            return carry

        lax.fori_loop(0, (n_chunks + 1) // 2, pair_body, 0)
        out_wait(0)
        out_wait(1)

    return edge_scores


def kernel(node_embeddings, edge_index, W_q, W_k):
    n_nodes = node_embeddings.shape[0]
    n_edges = edge_index.shape[1]
    project = _make_project(n_nodes, 2000)
    q_tab, k_tab = project(node_embeddings, W_q.T, W_k.T)
    edge_scores = _make_edge_scores(n_edges)
    ei = edge_index.astype(jnp.int32)
    return edge_scores(q_tab, k_tab, ei[0], ei[1])
